# Initial kernel scaffold; baseline (speedup 1.0000x reference)
import jax
import jax.numpy as jnp
from jax.experimental import pallas as pl


def kernel(h, e, edge_index, w, ne_W1, ne_b1, ne_W2, ne_b2, ee_W1, ee_b1, ee_W2, ee_b2, pe_W1, pe_b1, pe_W2, pe_b2, pn_W1, pn_b1, pn_W2, pn_b2, nd_W1, nd_b1, nd_W2, nd_b2, ed_W1, ed_b1, ed_W2, ed_b2):
    raise NotImplementedError("write your pallas kernel here")


# speedup vs baseline: 3.0926x; 3.0926x over previous
import functools

import jax
import jax.numpy as jnp
from jax import lax
from jax.experimental import pallas as pl
from jax.experimental.pallas import tpu as pltpu
from jax.experimental.pallas import tpu_sc as plsc

N = 10000
E = 320000
MH = 32
HID = 128
H2 = 2 * HID
OUT = 128

NW = 32
CH = 128
NCHUNK = E // CH
RPS = N // 16

F32 = jnp.float32


def _node_prep_body(h_ref, W1, b1, WA, bA, WB, bB, WP, bP,
                    hA_ref, hB_ref, xP_ref):
    u = jnp.maximum(
        jnp.dot(h_ref[...], W1[...], preferred_element_type=F32) + b1[...], 0.0)
    hA_ref[...] = jnp.dot(u, WA[...], preferred_element_type=F32) + bA[...]
    hB_ref[...] = jnp.dot(u, WB[...], preferred_element_type=F32) + bB[...]
    xP_ref[...] = jnp.dot(u, WP[...], preferred_element_type=F32) + bP[...]


def _edge_enc_body(e_ref, W1, b1, W2, b2, out_ref):
    u = jnp.maximum(
        jnp.dot(e_ref[...], W1[...], preferred_element_type=F32) + b1[...], 0.0)
    out_ref[...] = jnp.dot(u, W2[...], preferred_element_type=F32) + b2[...]


def _edge_main_body(epre_ref, gA_ref, gB_ref, w_ref, Me, ce, W2, b2,
                    eout_ref, rw_ref):
    r = jnp.maximum(epre_ref[...] + gA_ref[...] + gB_ref[...], 0.0)
    t = jnp.maximum(jnp.dot(r, Me[...], preferred_element_type=F32) + ce[...], 0.0)
    eout_ref[...] = jnp.dot(t, W2[...], preferred_element_type=F32) + b2[...]
    wcol = w_ref[...]
    rw = r * wcol
    pad = jnp.zeros((rw.shape[0], 15), F32)
    rw_ref[...] = jnp.concatenate([rw, wcol, pad], axis=-1)


def _node_out_body(xP_ref, g0_ref, g1_ref, Q, qv, Mn, cn, W2, b2, out_ref):
    g = g0_ref[...] + g1_ref[...]
    G = g[:, :MH]
    sw = g[:, MH:MH + 1]
    z = xP_ref[...] + jnp.dot(G, Q[...], preferred_element_type=F32) + sw * qv[...]
    t = jnp.maximum(
        jnp.dot(jnp.maximum(z, 0.0), Mn[...], preferred_element_type=F32) + cn[...],
        0.0)
    out_ref[...] = jnp.dot(t, W2[...], preferred_element_type=F32) + b2[...]


def _full(shape):
    nd = len(shape)
    return pl.BlockSpec(shape, lambda i, _nd=nd: (0,) * _nd)


def _rows(blk, width):
    return pl.BlockSpec((blk, width), lambda i: (i, 0))


_MESH = plsc.VectorSubcoreMesh(core_axis_name="c", subcore_axis_name="s")
_SC_PARAMS = pltpu.CompilerParams(use_tc_tiling_on_sc=False)


@functools.partial(
    pl.kernel,
    mesh=_MESH,
    out_type=(jax.ShapeDtypeStruct((E, MH), F32),
              jax.ShapeDtypeStruct((E, MH), F32)),
    scratch_types=[
        pltpu.VMEM((CH,), jnp.int32),
        pltpu.VMEM((CH,), jnp.int32),
        pltpu.VMEM((CH, MH), F32),
        pltpu.VMEM((CH, MH), F32),
        pltpu.SemaphoreType.DMA,
        pltpu.SemaphoreType.DMA,
    ],
    compiler_params=_SC_PARAMS,
)
def _sc_gather(hA_hbm, hB_hbm, src_hbm, dst_hbm, gA_hbm, gB_hbm,
               si, di, bufA, bufB, semA, semB):
    c = lax.axis_index("c")
    s = lax.axis_index("s")
    wid = s * 2 + c
    nch = (NCHUNK - wid + NW - 1) // NW

    def body(j, carry):
        base = (wid + j * NW) * CH
        pltpu.sync_copy(src_hbm.at[pl.ds(base, CH)], si)
        pltpu.sync_copy(dst_hbm.at[pl.ds(base, CH)], di)
        cpA = pltpu.async_copy(hA_hbm.at[si], bufA, semA)
        cpB = pltpu.async_copy(hB_hbm.at[di], bufB, semB)
        cpA.wait()
        cpB.wait()
        pltpu.sync_copy(bufA, gA_hbm.at[pl.ds(base, CH)])
        pltpu.sync_copy(bufB, gB_hbm.at[pl.ds(base, CH)])
        return carry

    lax.fori_loop(0, nch, body, 0)


@functools.partial(
    pl.kernel,
    mesh=_MESH,
    out_type=jax.ShapeDtypeStruct((2 * N, 48), F32),
    scratch_types=[
        pltpu.VMEM((CH,), jnp.int32),
        pltpu.VMEM((CH, 48), F32),
        pltpu.VMEM_SHARED((N, 48), F32),
    ],
    compiler_params=_SC_PARAMS,
)
def _sc_scatter(rw_hbm, dst_hbm, zero_hbm, out_hbm, idx_v, buf, shared):
    c = lax.axis_index("c")
    s = lax.axis_index("s")
    wid = s * 2 + c
    pltpu.sync_copy(zero_hbm.at[pl.ds(s * RPS, RPS)],
                    shared.at[pl.ds(s * RPS, RPS)])
    plsc.subcore_barrier()
    nch = (NCHUNK - wid + NW - 1) // NW

    def body(j, carry):
        base = (wid + j * NW) * CH
        pltpu.sync_copy(dst_hbm.at[pl.ds(base, CH)], idx_v)
        pltpu.sync_copy(rw_hbm.at[pl.ds(base, CH)], buf)
        pltpu.sync_copy(buf, shared.at[idx_v], add=True)
        return carry

    lax.fori_loop(0, nch, body, 0)
    plsc.subcore_barrier()
    pltpu.sync_copy(shared.at[pl.ds(s * RPS, RPS)],
                    out_hbm.at[pl.ds(c * N + s * RPS, RPS)])


def kernel(h, e, edge_index, w,
           ne_W1, ne_b1, ne_W2, ne_b2,
           ee_W1, ee_b1, ee_W2, ee_b2,
           pe_W1, pe_b1, pe_W2, pe_b2,
           pn_W1, pn_b1, pn_W2, pn_b2,
           nd_W1, nd_b1, nd_W2, nd_b2,
           ed_W1, ed_b1, ed_W2, ed_b2):
    src = edge_index[0].astype(jnp.int32)
    dst = edge_index[1].astype(jnp.int32)
    w2d = w[:, None].astype(F32)

    S_e = pe_W1[:HID] + pe_W1[HID:2 * HID]
    A = pe_W1[2 * HID:3 * HID] + pe_W1[3 * HID:4 * HID]
    B = pe_W1[4 * HID:5 * HID] + pe_W1[5 * HID:6 * HID]
    P = pn_W1[:HID] + pn_W1[HID:2 * HID]
    Rp = pn_W1[2 * HID:]

    WA = ne_W2 @ A
    bA = (ne_b2 @ A)[None, :]
    WB = ne_W2 @ B
    bB = (ne_b2 @ B)[None, :]
    WP = ne_W2 @ P
    bP = (ne_b2 @ P + pn_b1)[None, :]

    De = ee_W2 @ S_e
    de = (ee_b2 @ S_e + pe_b1)[None, :]

    Me = pe_W2 @ ed_W1
    ce = (pe_b2 @ ed_W1 + ed_b1)[None, :]
    Q = pe_W2 @ Rp
    qv = (pe_b2 @ Rp)[None, :]
    Mn = pn_W2 @ nd_W1
    cn = (pn_b2 @ nd_W1 + nd_b1)[None, :]

    ne_b1r = ne_b1[None, :]
    ee_b1r = ee_b1[None, :]
    nd_b2r = nd_b2[None, :]
    ed_b2r = ed_b2[None, :]

    BN = 1000
    hA, hB, xP = pl.pallas_call(
        _node_prep_body,
        grid=(N // BN,),
        in_specs=[_rows(BN, HID), _full((HID, MH)), _full((1, MH)),
                  _full((MH, MH)), _full((1, MH)), _full((MH, MH)),
                  _full((1, MH)), _full((MH, MH)), _full((1, MH))],
        out_specs=[_rows(BN, MH)] * 3,
        out_shape=[jax.ShapeDtypeStruct((N, MH), F32)] * 3,
    )(h, ne_W1, ne_b1r, WA, bA, WB, bB, WP, bP)

    BE = 4000
    e_pre = pl.pallas_call(
        _edge_enc_body,
        grid=(E // BE,),
        in_specs=[_rows(BE, 16), _full((16, MH)), _full((1, MH)),
                  _full((MH, MH)), _full((1, MH))],
        out_specs=_rows(BE, MH),
        out_shape=jax.ShapeDtypeStruct((E, MH), F32),
    )(e, ee_W1, ee_b1r, De, de)

    gA, gB = _sc_gather(hA, hB, src, dst)

    BM = 2000
    e_out, rw48 = pl.pallas_call(
        _edge_main_body,
        grid=(E // BM,),
        in_specs=[_rows(BM, MH), _rows(BM, MH), _rows(BM, MH), _rows(BM, 1),
                  _full((MH, MH)), _full((1, MH)),
                  _full((MH, OUT)), _full((1, OUT))],
        out_specs=[_rows(BM, OUT), _rows(BM, 48)],
        out_shape=[jax.ShapeDtypeStruct((E, OUT), F32),
                   jax.ShapeDtypeStruct((E, 48), F32)],
    )(e_pre, gA, gB, w2d, Me, ce, ed_W2, ed_b2r)

    zero48 = jnp.zeros((N, 48), F32)
    gp = _sc_scatter(rw48, dst, zero48)

    h_out = pl.pallas_call(
        _node_out_body,
        grid=(N // BN,),
        in_specs=[_rows(BN, MH),
                  pl.BlockSpec((BN, 48), lambda i: (i, 0)),
                  pl.BlockSpec((BN, 48), lambda i: (i + N // BN, 0)),
                  _full((MH, MH)), _full((1, MH)),
                  _full((MH, MH)), _full((1, MH)),
                  _full((MH, OUT)), _full((1, OUT))],
        out_specs=_rows(BN, OUT),
        out_shape=jax.ShapeDtypeStruct((N, OUT), F32),
    )(xP, gp, gp, Q, qv, Mn, cn, nd_W2, nd_b2r)

    return (h_out, e_out)


# speedup vs baseline: 3.3991x; 1.0991x over previous
import functools

import jax
import jax.numpy as jnp
from jax import lax
from jax.experimental import pallas as pl
from jax.experimental.pallas import tpu as pltpu
from jax.experimental.pallas import tpu_sc as plsc

N = 10000
E = 320000
MH = 32
HID = 128
H2 = 2 * HID
OUT = 128

NW = 32
CH = 128
NCHUNK = E // CH
RPS = N // 16

F32 = jnp.float32


def _node_prep_body(h_ref, W1, b1, WA, bA, WB, bB, WP, bP,
                    hA_ref, hB_ref, xP_ref):
    u = jnp.maximum(
        jnp.dot(h_ref[...], W1[...], preferred_element_type=F32) + b1[...], 0.0)
    hA_ref[...] = jnp.dot(u, WA[...], preferred_element_type=F32) + bA[...]
    hB_ref[...] = jnp.dot(u, WB[...], preferred_element_type=F32) + bB[...]
    xP_ref[...] = jnp.dot(u, WP[...], preferred_element_type=F32) + bP[...]


def _edge_main_body(e_ref, gA_ref, gB_ref, w_ref, eW1, eb1, De, de, Me, ce,
                    W2, b2, eout_ref, rw_ref):
    u = jnp.maximum(
        jnp.dot(e_ref[...], eW1[...], preferred_element_type=F32) + eb1[...], 0.0)
    epre = jnp.dot(u, De[...], preferred_element_type=F32) + de[...]
    r = jnp.maximum(epre + gA_ref[...] + gB_ref[...], 0.0)
    t = jnp.maximum(jnp.dot(r, Me[...], preferred_element_type=F32) + ce[...], 0.0)
    eout_ref[...] = jnp.dot(t, W2[...], preferred_element_type=F32) + b2[...]
    wcol = w_ref[...]
    rw = r * wcol
    pad = jnp.zeros((rw.shape[0], 15), F32)
    rw_ref[...] = jnp.concatenate([rw, wcol, pad], axis=-1)


def _node_out_body(xP_ref, g0_ref, g1_ref, Q, qv, Mn, cn, W2, b2, out_ref):
    g = g0_ref[...] + g1_ref[...]
    G = g[:, :MH]
    sw = g[:, MH:MH + 1]
    z = xP_ref[...] + jnp.dot(G, Q[...], preferred_element_type=F32) + sw * qv[...]
    t = jnp.maximum(
        jnp.dot(jnp.maximum(z, 0.0), Mn[...], preferred_element_type=F32) + cn[...],
        0.0)
    out_ref[...] = jnp.dot(t, W2[...], preferred_element_type=F32) + b2[...]


def _full(shape):
    nd = len(shape)
    return pl.BlockSpec(shape, lambda i, _nd=nd: (0,) * _nd)


def _rows(blk, width):
    return pl.BlockSpec((blk, width), lambda i: (i, 0))


_MESH = plsc.VectorSubcoreMesh(core_axis_name="c", subcore_axis_name="s")
_SC_PARAMS = pltpu.CompilerParams(use_tc_tiling_on_sc=False)


@functools.partial(
    pl.kernel,
    mesh=_MESH,
    out_type=(jax.ShapeDtypeStruct((E, MH), F32),
              jax.ShapeDtypeStruct((E, MH), F32)),
    scratch_types=[
        pltpu.VMEM((CH,), jnp.int32),
        pltpu.VMEM((CH,), jnp.int32),
        pltpu.VMEM((CH, MH), F32),
        pltpu.VMEM((CH, MH), F32),
        pltpu.SemaphoreType.DMA,
        pltpu.SemaphoreType.DMA,
    ],
    compiler_params=_SC_PARAMS,
)
def _sc_gather(hA_hbm, hB_hbm, src_hbm, dst_hbm, gA_hbm, gB_hbm,
               si, di, bufA, bufB, semA, semB):
    c = lax.axis_index("c")
    s = lax.axis_index("s")
    wid = s * 2 + c
    nch = (NCHUNK - wid + NW - 1) // NW

    def body(j, carry):
        base = (wid + j * NW) * CH
        pltpu.sync_copy(src_hbm.at[pl.ds(base, CH)], si)
        pltpu.sync_copy(dst_hbm.at[pl.ds(base, CH)], di)
        cpA = pltpu.async_copy(hA_hbm.at[si], bufA, semA)
        cpB = pltpu.async_copy(hB_hbm.at[di], bufB, semB)
        cpA.wait()
        cpB.wait()
        pltpu.sync_copy(bufA, gA_hbm.at[pl.ds(base, CH)])
        pltpu.sync_copy(bufB, gB_hbm.at[pl.ds(base, CH)])
        return carry

    lax.fori_loop(0, nch, body, 0)


@functools.partial(
    pl.kernel,
    mesh=_MESH,
    out_type=jax.ShapeDtypeStruct((2 * N, 48), F32),
    scratch_types=[
        pltpu.VMEM((CH,), jnp.int32),
        pltpu.VMEM((CH, 48), F32),
        pltpu.VMEM_SHARED((N, 48), F32),
    ],
    compiler_params=_SC_PARAMS,
)
def _sc_scatter(rw_hbm, dst_hbm, zero_hbm, out_hbm, idx_v, buf, shared):
    c = lax.axis_index("c")
    s = lax.axis_index("s")
    wid = s * 2 + c
    pltpu.sync_copy(zero_hbm.at[pl.ds(s * RPS, RPS)],
                    shared.at[pl.ds(s * RPS, RPS)])
    plsc.subcore_barrier()
    nch = (NCHUNK - wid + NW - 1) // NW

    def body(j, carry):
        base = (wid + j * NW) * CH
        pltpu.sync_copy(dst_hbm.at[pl.ds(base, CH)], idx_v)
        pltpu.sync_copy(rw_hbm.at[pl.ds(base, CH)], buf)
        pltpu.sync_copy(buf, shared.at[idx_v], add=True)
        return carry

    lax.fori_loop(0, nch, body, 0)
    plsc.subcore_barrier()
    pltpu.sync_copy(shared.at[pl.ds(s * RPS, RPS)],
                    out_hbm.at[pl.ds(c * N + s * RPS, RPS)])


def kernel(h, e, edge_index, w,
           ne_W1, ne_b1, ne_W2, ne_b2,
           ee_W1, ee_b1, ee_W2, ee_b2,
           pe_W1, pe_b1, pe_W2, pe_b2,
           pn_W1, pn_b1, pn_W2, pn_b2,
           nd_W1, nd_b1, nd_W2, nd_b2,
           ed_W1, ed_b1, ed_W2, ed_b2):
    src = edge_index[0].astype(jnp.int32)
    dst = edge_index[1].astype(jnp.int32)
    w2d = w[:, None].astype(F32)

    S_e = pe_W1[:HID] + pe_W1[HID:2 * HID]
    A = pe_W1[2 * HID:3 * HID] + pe_W1[3 * HID:4 * HID]
    B = pe_W1[4 * HID:5 * HID] + pe_W1[5 * HID:6 * HID]
    P = pn_W1[:HID] + pn_W1[HID:2 * HID]
    Rp = pn_W1[2 * HID:]

    WA = ne_W2 @ A
    bA = (ne_b2 @ A)[None, :]
    WB = ne_W2 @ B
    bB = (ne_b2 @ B)[None, :]
    WP = ne_W2 @ P
    bP = (ne_b2 @ P + pn_b1)[None, :]

    De = ee_W2 @ S_e
    de = (ee_b2 @ S_e + pe_b1)[None, :]

    Me = pe_W2 @ ed_W1
    ce = (pe_b2 @ ed_W1 + ed_b1)[None, :]
    Q = pe_W2 @ Rp
    qv = (pe_b2 @ Rp)[None, :]
    Mn = pn_W2 @ nd_W1
    cn = (pn_b2 @ nd_W1 + nd_b1)[None, :]

    ne_b1r = ne_b1[None, :]
    ee_b1r = ee_b1[None, :]
    nd_b2r = nd_b2[None, :]
    ed_b2r = ed_b2[None, :]

    BN = 1000
    hA, hB, xP = pl.pallas_call(
        _node_prep_body,
        grid=(N // BN,),
        in_specs=[_rows(BN, HID), _full((HID, MH)), _full((1, MH)),
                  _full((MH, MH)), _full((1, MH)), _full((MH, MH)),
                  _full((1, MH)), _full((MH, MH)), _full((1, MH))],
        out_specs=[_rows(BN, MH)] * 3,
        out_shape=[jax.ShapeDtypeStruct((N, MH), F32)] * 3,
    )(h, ne_W1, ne_b1r, WA, bA, WB, bB, WP, bP)

    gA, gB = _sc_gather(hA, hB, src, dst)

    BM = 2000
    e_out, rw48 = pl.pallas_call(
        _edge_main_body,
        grid=(E // BM,),
        in_specs=[_rows(BM, 16), _rows(BM, MH), _rows(BM, MH), _rows(BM, 1),
                  _full((16, MH)), _full((1, MH)),
                  _full((MH, MH)), _full((1, MH)),
                  _full((MH, MH)), _full((1, MH)),
                  _full((MH, OUT)), _full((1, OUT))],
        out_specs=[_rows(BM, OUT), _rows(BM, 48)],
        out_shape=[jax.ShapeDtypeStruct((E, OUT), F32),
                   jax.ShapeDtypeStruct((E, 48), F32)],
    )(e, gA, gB, w2d, ee_W1, ee_b1r, De, de, Me, ce, ed_W2, ed_b2r)

    zero48 = jnp.zeros((N, 48), F32)
    gp = _sc_scatter(rw48, dst, zero48)

    h_out = pl.pallas_call(
        _node_out_body,
        grid=(N // BN,),
        in_specs=[_rows(BN, MH),
                  pl.BlockSpec((BN, 48), lambda i: (i, 0)),
                  pl.BlockSpec((BN, 48), lambda i: (i + N // BN, 0)),
                  _full((MH, MH)), _full((1, MH)),
                  _full((MH, MH)), _full((1, MH)),
                  _full((MH, OUT)), _full((1, OUT))],
        out_specs=_rows(BN, OUT),
        out_shape=jax.ShapeDtypeStruct((N, OUT), F32),
    )(xP, gp, gp, Q, qv, Mn, cn, nd_W2, nd_b2r)

    return (h_out, e_out)


# speedup vs baseline: 3.4926x; 1.0275x over previous
import functools

import jax
import jax.numpy as jnp
from jax import lax
from jax.experimental import pallas as pl
from jax.experimental.pallas import tpu as pltpu
from jax.experimental.pallas import tpu_sc as plsc

N = 10000
E = 320000
MH = 32
HID = 128
H2 = 2 * HID
OUT = 128

NW = 32
SUB = 100
NSUB = 4
CHO = SUB * NSUB
EPW = E // NW
NCHO = EPW // CHO
RPW = EPW // SUB
RPS = N // 16

F32 = jnp.float32


def _node_prep_body(h_ref, W1, b1, WA, bA, WB, bB, WP, bP,
                    hA_ref, hB_ref, xP_ref):
    u = jnp.maximum(
        jnp.dot(h_ref[...], W1[...], preferred_element_type=F32) + b1[...], 0.0)
    hA_ref[...] = jnp.dot(u, WA[...], preferred_element_type=F32) + bA[...]
    hB_ref[...] = jnp.dot(u, WB[...], preferred_element_type=F32) + bB[...]
    xP_ref[...] = jnp.dot(u, WP[...], preferred_element_type=F32) + bP[...]


def _edge_main_body(e_ref, gA_ref, gB_ref, w_ref, eW1, eb1, De, de, Me, ce,
                    W2, b2, eout_ref, rw_ref):
    u = jnp.maximum(
        jnp.dot(e_ref[...], eW1[...], preferred_element_type=F32) + eb1[...], 0.0)
    epre = jnp.dot(u, De[...], preferred_element_type=F32) + de[...]
    r = jnp.maximum(epre + gA_ref[...] + gB_ref[...], 0.0)
    t = jnp.maximum(jnp.dot(r, Me[...], preferred_element_type=F32) + ce[...], 0.0)
    eout_ref[...] = jnp.dot(t, W2[...], preferred_element_type=F32) + b2[...]
    wcol = w_ref[...]
    rw = r * wcol
    pad = jnp.zeros((rw.shape[0], 15), F32)
    rw_ref[...] = jnp.concatenate([rw, wcol, pad], axis=-1)


def _node_out_body(xP_ref, g0_ref, g1_ref, Q, qv, Mn, cn, W2, b2, out_ref):
    g = g0_ref[...] + g1_ref[...]
    G = g[:, :MH]
    sw = g[:, MH:MH + 1]
    z = xP_ref[...] + jnp.dot(G, Q[...], preferred_element_type=F32) + sw * qv[...]
    t = jnp.maximum(
        jnp.dot(jnp.maximum(z, 0.0), Mn[...], preferred_element_type=F32) + cn[...],
        0.0)
    out_ref[...] = jnp.dot(t, W2[...], preferred_element_type=F32) + b2[...]


def _full(shape):
    nd = len(shape)
    return pl.BlockSpec(shape, lambda i, _nd=nd: (0,) * _nd)


def _rows(blk, width):
    return pl.BlockSpec((blk, width), lambda i: (i, 0))


_MESH = plsc.VectorSubcoreMesh(core_axis_name="c", subcore_axis_name="s")
_SC_PARAMS = pltpu.CompilerParams(use_tc_tiling_on_sc=False)


@functools.partial(
    pl.kernel,
    mesh=_MESH,
    out_type=(jax.ShapeDtypeStruct((E, MH), F32),
              jax.ShapeDtypeStruct((E, MH), F32)),
    scratch_types=[
        pltpu.VMEM((RPW // NCHO, SUB), jnp.int32),
        pltpu.VMEM((RPW // NCHO, SUB), jnp.int32),
        pltpu.VMEM((CHO, MH), F32),
        pltpu.VMEM((CHO, MH), F32),
        pltpu.SemaphoreType.DMA,
        pltpu.SemaphoreType.DMA,
    ],
    compiler_params=_SC_PARAMS,
)
def _sc_gather(hA_hbm, hB_hbm, src3_hbm, dst3_hbm, gA_hbm, gB_hbm,
               ibs, ibd, bufA, bufB, semA, semB):
    c = lax.axis_index("c")
    s = lax.axis_index("s")
    wid = s * 2 + c

    def body(j, carry):
        erow = wid * RPW + j * NSUB
        base = wid * EPW + j * CHO
        pltpu.sync_copy(src3_hbm.at[pl.ds(erow, NSUB)], ibs)
        pltpu.sync_copy(dst3_hbm.at[pl.ds(erow, NSUB)], ibd)
        cps = []
        for k in range(NSUB):
            cps.append(pltpu.async_copy(
                hA_hbm.at[ibs.at[k]], bufA.at[pl.ds(k * SUB, SUB)], semA))
            cps.append(pltpu.async_copy(
                hB_hbm.at[ibd.at[k]], bufB.at[pl.ds(k * SUB, SUB)], semB))
        for cp in cps:
            cp.wait()
        pltpu.sync_copy(bufA, gA_hbm.at[pl.ds(base, CHO)])
        pltpu.sync_copy(bufB, gB_hbm.at[pl.ds(base, CHO)])
        return carry

    lax.fori_loop(0, NCHO, body, 0)


@functools.partial(
    pl.kernel,
    mesh=_MESH,
    out_type=jax.ShapeDtypeStruct((2 * N, 48), F32),
    scratch_types=[
        pltpu.VMEM((RPW // NCHO, SUB), jnp.int32),
        pltpu.VMEM((CHO, 48), F32),
        pltpu.VMEM_SHARED((N, 48), F32),
        pltpu.SemaphoreType.DMA,
    ],
    compiler_params=_SC_PARAMS,
)
def _sc_scatter(rw_hbm, dst3_hbm, zero_hbm, out_hbm, ibd, buf, shared, sem):
    c = lax.axis_index("c")
    s = lax.axis_index("s")
    wid = s * 2 + c
    pltpu.sync_copy(zero_hbm.at[pl.ds(s * RPS, RPS)],
                    shared.at[pl.ds(s * RPS, RPS)])
    plsc.subcore_barrier()

    def body(j, carry):
        erow = wid * RPW + j * NSUB
        base = wid * EPW + j * CHO
        pltpu.sync_copy(dst3_hbm.at[pl.ds(erow, NSUB)], ibd)
        pltpu.sync_copy(rw_hbm.at[pl.ds(base, CHO)], buf)
        cps = []
        for k in range(NSUB):
            cps.append(pltpu.async_copy(
                buf.at[pl.ds(k * SUB, SUB)], shared.at[ibd.at[k]], sem,
                add=True))
        for cp in cps:
            cp.wait()
        return carry

    lax.fori_loop(0, NCHO, body, 0)
    plsc.subcore_barrier()
    pltpu.sync_copy(shared.at[pl.ds(s * RPS, RPS)],
                    out_hbm.at[pl.ds(c * N + s * RPS, RPS)])


def kernel(h, e, edge_index, w,
           ne_W1, ne_b1, ne_W2, ne_b2,
           ee_W1, ee_b1, ee_W2, ee_b2,
           pe_W1, pe_b1, pe_W2, pe_b2,
           pn_W1, pn_b1, pn_W2, pn_b2,
           nd_W1, nd_b1, nd_W2, nd_b2,
           ed_W1, ed_b1, ed_W2, ed_b2):
    src = edge_index[0].astype(jnp.int32)
    dst = edge_index[1].astype(jnp.int32)
    src3 = src.reshape(E // SUB, SUB)
    dst3 = dst.reshape(E // SUB, SUB)
    w2d = w[:, None].astype(F32)

    S_e = pe_W1[:HID] + pe_W1[HID:2 * HID]
    A = pe_W1[2 * HID:3 * HID] + pe_W1[3 * HID:4 * HID]
    B = pe_W1[4 * HID:5 * HID] + pe_W1[5 * HID:6 * HID]
    P = pn_W1[:HID] + pn_W1[HID:2 * HID]
    Rp = pn_W1[2 * HID:]

    WA = ne_W2 @ A
    bA = (ne_b2 @ A)[None, :]
    WB = ne_W2 @ B
    bB = (ne_b2 @ B)[None, :]
    WP = ne_W2 @ P
    bP = (ne_b2 @ P + pn_b1)[None, :]

    De = ee_W2 @ S_e
    de = (ee_b2 @ S_e + pe_b1)[None, :]

    Me = pe_W2 @ ed_W1
    ce = (pe_b2 @ ed_W1 + ed_b1)[None, :]
    Q = pe_W2 @ Rp
    qv = (pe_b2 @ Rp)[None, :]
    Mn = pn_W2 @ nd_W1
    cn = (pn_b2 @ nd_W1 + nd_b1)[None, :]

    ne_b1r = ne_b1[None, :]
    ee_b1r = ee_b1[None, :]
    nd_b2r = nd_b2[None, :]
    ed_b2r = ed_b2[None, :]

    BN = 1000
    hA, hB, xP = pl.pallas_call(
        _node_prep_body,
        grid=(N // BN,),
        in_specs=[_rows(BN, HID), _full((HID, MH)), _full((1, MH)),
                  _full((MH, MH)), _full((1, MH)), _full((MH, MH)),
                  _full((1, MH)), _full((MH, MH)), _full((1, MH))],
        out_specs=[_rows(BN, MH)] * 3,
        out_shape=[jax.ShapeDtypeStruct((N, MH), F32)] * 3,
    )(h, ne_W1, ne_b1r, WA, bA, WB, bB, WP, bP)

    gA, gB = _sc_gather(hA, hB, src3, dst3)

    BM = 2000
    e_out, rw48 = pl.pallas_call(
        _edge_main_body,
        grid=(E // BM,),
        in_specs=[_rows(BM, 16), _rows(BM, MH), _rows(BM, MH), _rows(BM, 1),
                  _full((16, MH)), _full((1, MH)),
                  _full((MH, MH)), _full((1, MH)),
                  _full((MH, MH)), _full((1, MH)),
                  _full((MH, OUT)), _full((1, OUT))],
        out_specs=[_rows(BM, OUT), _rows(BM, 48)],
        out_shape=[jax.ShapeDtypeStruct((E, OUT), F32),
                   jax.ShapeDtypeStruct((E, 48), F32)],
    )(e, gA, gB, w2d, ee_W1, ee_b1r, De, de, Me, ce, ed_W2, ed_b2r)

    zero48 = jnp.zeros((N, 48), F32)
    gp = _sc_scatter(rw48, dst3, zero48)

    h_out = pl.pallas_call(
        _node_out_body,
        grid=(N // BN,),
        in_specs=[_rows(BN, MH),
                  pl.BlockSpec((BN, 48), lambda i: (i, 0)),
                  pl.BlockSpec((BN, 48), lambda i: (i + N // BN, 0)),
                  _full((MH, MH)), _full((1, MH)),
                  _full((MH, MH)), _full((1, MH)),
                  _full((MH, OUT)), _full((1, OUT))],
        out_specs=_rows(BN, OUT),
        out_shape=jax.ShapeDtypeStruct((N, OUT), F32),
    )(xP, gp, gp, Q, qv, Mn, cn, nd_W2, nd_b2r)

    return (h_out, e_out)
